# baseline (speedup 1.0000x reference)
import functools

import jax
import jax.numpy as jnp
from jax import lax
from jax.experimental import pallas as pl
from jax.experimental.pallas import tpu as pltpu
from jax.experimental.pallas import tpu_sc as plsc

D_MODEL = 64
DPAD = 128
SCALE = 8.0
NC, NS, L = 2, 16, 16
NW = NC * NS

XR, XC = 4096, 50
B = XR * XC
XRPW = XR // NW
BPW = B // NW
CXR = 4
CH = CXR * XC
NCH = XRPW // CXR
NBUF = 2

_mesh = plsc.VectorSubcoreMesh(core_axis_name="c", subcore_axis_name="s")


@functools.partial(
    pl.kernel,
    out_type=jax.ShapeDtypeStruct((XR, XC, D_MODEL), jnp.float32),
    mesh=_mesh,
    scratch_types=[
        pltpu.VMEM((BPW,), jnp.int32),
        pltpu.VMEM((NBUF, CH, DPAD), jnp.float32),
        pltpu.VMEM((NBUF, CXR, XC, D_MODEL), jnp.float32),
        pltpu.SemaphoreType.DMA((NBUF,)),
        pltpu.SemaphoreType.DMA((NBUF,)),
    ],
    compiler_params=pltpu.CompilerParams(use_tc_tiling_on_sc=True),
)
def _emb_lookup(x_hbm, table_hbm, out_hbm, idx_v, rows_v, obuf, gsem, ssem):
    wid = lax.axis_index("s") * NC + lax.axis_index("c")
    base = wid * BPW
    xrbase = wid * XRPW
    pltpu.sync_copy(x_hbm.at[pl.ds(base, BPW)], idx_v)

    def start_gather(g, b):
        pltpu.async_copy(
            table_hbm.at[idx_v.at[pl.ds(g * CH, CH)]], rows_v.at[b],
            gsem.at[b])

    def wait_gather(b):
        pltpu.make_async_copy(
            table_hbm.at[idx_v.at[pl.ds(0, CH)]], rows_v.at[b],
            gsem.at[b]).wait()

    def start_scatter(g, b):
        for xr in range(CXR):
            pltpu.async_copy(
                obuf.at[b].at[xr],
                out_hbm.at[xrbase + g * CXR + xr], ssem.at[b])

    def wait_scatter(b):
        for _ in range(CXR):
            pltpu.make_async_copy(
                obuf.at[b].at[0],
                out_hbm.at[xrbase], ssem.at[b]).wait()

    for b in range(NBUF - 1):
        start_gather(b, b)

    @pl.loop(0, NCH, step=NBUF)
    def _group(g0):
        for j in range(NBUF):
            g = g0 + j
            wait_gather(j)

            @pl.when(g >= NBUF)
            def _():
                wait_scatter(j)
            rv = rows_v.at[j]
            for xr in range(CXR):
                ov = obuf.at[j].at[xr]

                @pl.loop(0, XC, unroll=5)
                def _row(r):
                    for c in range(D_MODEL // L):
                        sl = pl.ds(c * L, L)
                        ov[r, sl] = rv[xr * XC + r, sl] * SCALE

            start_scatter(g, j)
            h = g + NBUF - 1
            bh = (j + NBUF - 1) % NBUF

            @pl.when(h < NCH)
            def _():
                start_gather(h, bh)

    for b in range(NBUF):
        wait_scatter(b)


def kernel(x, table):
    tab_pad = jnp.pad(table, ((0, 0), (0, DPAD - D_MODEL)))
    return _emb_lookup(x.reshape(-1), tab_pad)


# speedup vs baseline: 1.1392x; 1.1392x over previous
import functools

import jax
import jax.numpy as jnp
from jax import lax
from jax.experimental import pallas as pl
from jax.experimental.pallas import tpu as pltpu
from jax.experimental.pallas import tpu_sc as plsc

D_MODEL = 64
SCALE = 8.0
NC, NS, L = 2, 16, 16
NW = NC * NS

XR, XC = 4096, 50
B = XR * XC
XRPW = XR // NW
BPW = B // NW
CXR = 4
CH = CXR * XC
NCH = XRPW // CXR
NBUF = 4

_mesh = plsc.VectorSubcoreMesh(core_axis_name="c", subcore_axis_name="s")


@functools.partial(
    pl.kernel,
    out_type=jax.ShapeDtypeStruct((XR, XC, D_MODEL), jnp.float32),
    mesh=_mesh,
    scratch_types=[
        pltpu.VMEM((BPW,), jnp.int32),
        pltpu.VMEM((NBUF, CH, D_MODEL), jnp.float32),
        pltpu.SemaphoreType.DMA((NBUF,)),
        pltpu.SemaphoreType.DMA((NBUF,)),
    ],
    compiler_params=pltpu.CompilerParams(use_tc_tiling_on_sc=False),
)
def _emb_lookup(x_hbm, table_hbm, out_hbm, idx_v, rows_v, gsem, ssem):
    wid = lax.axis_index("s") * NC + lax.axis_index("c")
    base = wid * BPW
    xrbase = wid * XRPW
    pltpu.sync_copy(x_hbm.at[pl.ds(base, BPW)], idx_v)

    def start_gather(g, b):
        pltpu.async_copy(
            table_hbm.at[idx_v.at[pl.ds(g * CH, CH)]], rows_v.at[b],
            gsem.at[b])

    def wait_gather(b):
        pltpu.make_async_copy(
            table_hbm.at[idx_v.at[pl.ds(0, CH)]], rows_v.at[b],
            gsem.at[b]).wait()

    def start_scatter(g, b):
        for xr in range(CXR):
            pltpu.async_copy(
                rows_v.at[b].at[pl.ds(xr * XC, XC)],
                out_hbm.at[xrbase + g * CXR + xr], ssem.at[b])

    def wait_scatter(b):
        for _ in range(CXR):
            pltpu.make_async_copy(
                rows_v.at[b].at[pl.ds(0, XC)],
                out_hbm.at[xrbase], ssem.at[b]).wait()

    for b in range(NBUF - 1):
        start_gather(b, b)

    @pl.loop(0, NCH, step=NBUF)
    def _group(g0):
        for j in range(NBUF):
            g = g0 + j
            wait_gather(j)
            rv = rows_v.at[j]

            @pl.loop(0, CH, unroll=8)
            def _row(r):
                for c in range(D_MODEL // L):
                    sl = pl.ds(c * L, L)
                    rv[r, sl] = rv[r, sl] * SCALE

            start_scatter(g, j)
            h = g + NBUF - 1
            bh = (j + NBUF - 1) % NBUF

            @pl.when(h < NCH)
            def _():
                @pl.when(g >= 1)
                def _():
                    wait_scatter(bh)
                start_gather(h, bh)

    for b in range(NBUF):
        wait_scatter(b)


def kernel(x, table):
    return _emb_lookup(x.reshape(-1), table)
